# baseline (speedup 1.0000x reference)
import functools

import jax
import jax.numpy as jnp
from jax import lax
from jax.experimental import pallas as pl
from jax.experimental.pallas import tpu as pltpu
from jax.experimental.pallas import tpu_sc as plsc

_D = 64


@functools.lru_cache(maxsize=None)
def _build(batch: int):
    info = plsc.get_sparse_core_info()
    nw = info.num_cores * info.num_subcores
    bpw = batch // nw
    mesh = plsc.VectorSubcoreMesh(core_axis_name="c", subcore_axis_name="s")

    @functools.partial(
        pl.kernel,
        mesh=mesh,
        out_type=(
            jax.ShapeDtypeStruct((_D, batch), jnp.float32),
            jax.ShapeDtypeStruct((_D, batch), jnp.float32),
        ),
        scratch_types=[
            pltpu.VMEM((bpw,), jnp.int32),
            pltpu.VMEM((bpw,), jnp.int32),
            pltpu.VMEM((_D, bpw), jnp.float32),
            pltpu.VMEM((_D, bpw), jnp.float32),
            pltpu.SemaphoreType.DMA,
        ],
        compiler_params=pltpu.CompilerParams(use_tc_tiling_on_sc=False),
    )
    def od_gather(ori_hbm, dest_hbm, tbl_t_hbm, out_o_hbm, out_d_hbm,
                  idx_o, idx_d, cols_o, cols_d, sem):
        wid = lax.axis_index("s") * info.num_cores + lax.axis_index("c")
        base = wid * bpw
        pltpu.sync_copy(ori_hbm.at[pl.ds(base, bpw)], idx_o)
        pltpu.sync_copy(dest_hbm.at[pl.ds(base, bpw)], idx_d)

        for d in range(_D):
            pltpu.async_copy(
                tbl_t_hbm.at[d].at[idx_o], cols_o.at[d], sem)
            pltpu.async_copy(
                tbl_t_hbm.at[d].at[idx_d], cols_d.at[d], sem)

        pltpu.make_async_copy(
            out_o_hbm.at[:, pl.ds(0, bpw)], cols_o, sem).wait()
        pltpu.make_async_copy(
            out_o_hbm.at[:, pl.ds(0, bpw)], cols_d, sem).wait()

        pltpu.sync_copy(cols_o, out_o_hbm.at[:, pl.ds(base, bpw)])
        pltpu.sync_copy(cols_d, out_d_hbm.at[:, pl.ds(base, bpw)])

    return od_gather


def kernel(ori, dest, table):
    batch, = ori.shape
    out_o_t, out_d_t = _build(batch)(ori, dest, table.T)
    return (out_o_t.T, out_d_t.T)
